# baseline (speedup 1.0000x reference)
import jax
import jax.numpy as jnp
from jax import lax
from jax.experimental import pallas as pl
from jax.experimental.pallas import tpu as pltpu
from jax.experimental.pallas import tpu_sc as plsc

_EMB_DIM = 32
_NUM_CODES = 8192
_N_ROWS = 4096
_BETA = 0.25

_NT = 2048
_KT = 512


def _argmin_body(x_ref, e_ref, idx_ref, loss_ref, se_ref):
    i = pl.program_id(0)

    @pl.when(i == 0)
    def _se():
        e = e_ref[...]
        se_ref[...] = 0.25 * jnp.sum(e * e, axis=0, keepdims=True)

    x = x_ref[...]
    sx = jnp.sum(x * x, axis=1, keepdims=True)
    run_min = jnp.full((_NT, 1), jnp.inf, jnp.float32)
    run_idx = jnp.zeros((_NT, 1), jnp.float32)
    iif = lax.broadcasted_iota(jnp.int32, (_NT, _KT), 1).astype(jnp.float32)
    for c in range(_NUM_CODES // _KT):
        eb = e_ref[:, c * _KT:(c + 1) * _KT]
        mm2 = lax.dot_general(x, eb, (((1,), (0,)), ((), ())),
                              preferred_element_type=jnp.float32)
        d = sx - mm2
        d = d + se_ref[:, c * _KT:(c + 1) * _KT]
        m = jnp.min(d, axis=1, keepdims=True)
        cand = jnp.min(jnp.where(d == m, iif, float(_KT)), axis=1,
                       keepdims=True) + float(c * _KT)
        better = m < run_min
        run_idx = jnp.where(better, cand, run_idx)
        run_min = jnp.where(better, m, run_min)
    idx_ref[0, 0, :] = run_idx[:, 0].astype(jnp.int32)

    @pl.when(i == 0)
    def _init():
        loss_ref[...] = jnp.zeros((1, 1), jnp.float32)

    loss_ref[...] = loss_ref[...] + jnp.sum(run_min)

    @pl.when(i == pl.num_programs(0) - 1)
    def _scale():
        loss_ref[...] = loss_ref[...] * (_BETA / (16 * 8192))


_N_TILES = _N_ROWS // _NT

_argmin_call = pl.pallas_call(
    _argmin_body,
    grid=(_N_TILES,),
    in_specs=[
        pl.BlockSpec((_NT, _EMB_DIM), lambda i: (i, 0)),
        pl.BlockSpec((_EMB_DIM, _NUM_CODES), lambda i: (0, 0)),
    ],
    out_specs=[
        pl.BlockSpec((1, 1, _NT), lambda i: (i, 0, 0)),
        pl.BlockSpec((1, 1), lambda i: (0, 0)),
    ],
    out_shape=[
        jax.ShapeDtypeStruct((_N_TILES, 1, _NT), jnp.int32),
        jax.ShapeDtypeStruct((1, 1), jnp.float32),
    ],
    scratch_shapes=[pltpu.VMEM((1, _NUM_CODES), jnp.float32)],
)


_D_PAD = 128


def _sc_gather_body(et_hbm, idx_hbm, out_hbm, idx_v, rows_v, sem):
    info = plsc.get_sparse_core_info()
    rows_per_w = _N_ROWS // (info.num_cores * info.num_subcores)
    wid = lax.axis_index("s") * info.num_cores + lax.axis_index("c")
    base = wid * rows_per_w
    pltpu.sync_copy(idx_hbm.at[pl.ds(base, rows_per_w)], idx_v)
    pltpu.async_copy(et_hbm.at[idx_v], rows_v, sem).wait()
    pltpu.sync_copy(rows_v, out_hbm.at[pl.ds(base, rows_per_w)])


def _make_sc_gather():
    info = plsc.get_sparse_core_info()
    rows_per_w = _N_ROWS // (info.num_cores * info.num_subcores)
    mesh = plsc.VectorSubcoreMesh(core_axis_name="c", subcore_axis_name="s")
    return pl.kernel(
        _sc_gather_body,
        out_type=jax.ShapeDtypeStruct((_N_ROWS, _D_PAD), jnp.float32),
        mesh=mesh,
        scratch_types=[
            pltpu.VMEM((rows_per_w,), jnp.int32),
            pltpu.VMEM((rows_per_w, _D_PAD), jnp.float32),
            pltpu.SemaphoreType.DMA,
        ],
    )


def kernel(inputs, embeddings):
    x = inputs.reshape(_N_ROWS, _EMB_DIM)
    idx3, loss2 = _argmin_call(x, embeddings + embeddings)
    idx = idx3.reshape(_N_ROWS)
    et = jnp.pad(embeddings.T, ((0, 0), (0, _D_PAD - _EMB_DIM)))
    q = _make_sc_gather()(et, idx)
    out = q[:, :_EMB_DIM].reshape(16, 8192, 1)
    return (out, loss2[0, 0], 0.0)


# speedup vs baseline: 1.0113x; 1.0113x over previous
import jax
import jax.numpy as jnp
from jax import lax
from jax.experimental import pallas as pl
from jax.experimental.pallas import tpu as pltpu
from jax.experimental.pallas import tpu_sc as plsc

_EMB_DIM = 32
_NUM_CODES = 8192
_N_ROWS = 4096
_BETA = 0.25

_NT = 2048
_KT = 1024


def _argmin_body(x_ref, e_ref, idx_ref, loss_ref, se_ref):
    i = pl.program_id(0)

    @pl.when(i == 0)
    def _se():
        e = e_ref[...]
        se_ref[...] = 0.25 * jnp.sum(e * e, axis=0, keepdims=True)

    x = x_ref[...]
    sx = jnp.sum(x * x, axis=1, keepdims=True)
    run_min = jnp.full((_NT, 1), jnp.inf, jnp.float32)
    run_idx = jnp.zeros((_NT, 1), jnp.float32)
    iif = lax.broadcasted_iota(jnp.int32, (_NT, _KT), 1).astype(jnp.float32)
    for c in range(_NUM_CODES // _KT):
        eb = e_ref[:, c * _KT:(c + 1) * _KT]
        mm2 = lax.dot_general(x, eb, (((1,), (0,)), ((), ())),
                              preferred_element_type=jnp.float32)
        d = sx - mm2
        d = d + se_ref[:, c * _KT:(c + 1) * _KT]
        m = jnp.min(d, axis=1, keepdims=True)
        cand = jnp.min(jnp.where(d == m, iif, float(_KT)), axis=1,
                       keepdims=True) + float(c * _KT)
        better = m < run_min
        run_idx = jnp.where(better, cand, run_idx)
        run_min = jnp.where(better, m, run_min)
    idx_ref[0, 0, :] = run_idx[:, 0].astype(jnp.int32)

    @pl.when(i == 0)
    def _init():
        loss_ref[...] = jnp.zeros((1, 1), jnp.float32)

    loss_ref[...] = loss_ref[...] + jnp.sum(run_min)

    @pl.when(i == pl.num_programs(0) - 1)
    def _scale():
        loss_ref[...] = loss_ref[...] * (_BETA / (16 * 8192))


_N_TILES = _N_ROWS // _NT

_argmin_call = pl.pallas_call(
    _argmin_body,
    grid=(_N_TILES,),
    in_specs=[
        pl.BlockSpec((_NT, _EMB_DIM), lambda i: (i, 0)),
        pl.BlockSpec((_EMB_DIM, _NUM_CODES), lambda i: (0, 0)),
    ],
    out_specs=[
        pl.BlockSpec((1, 1, _NT), lambda i: (i, 0, 0)),
        pl.BlockSpec((1, 1), lambda i: (0, 0)),
    ],
    out_shape=[
        jax.ShapeDtypeStruct((_N_TILES, 1, _NT), jnp.int32),
        jax.ShapeDtypeStruct((1, 1), jnp.float32),
    ],
    scratch_shapes=[pltpu.VMEM((1, _NUM_CODES), jnp.float32)],
)


_D_PAD = 128


def _sc_gather_body(et_hbm, idx_hbm, out_hbm, idx_v, rows_v, sem):
    info = plsc.get_sparse_core_info()
    rows_per_w = _N_ROWS // (info.num_cores * info.num_subcores)
    wid = lax.axis_index("s") * info.num_cores + lax.axis_index("c")
    base = wid * rows_per_w
    pltpu.sync_copy(idx_hbm.at[pl.ds(base, rows_per_w)], idx_v)
    pltpu.async_copy(et_hbm.at[idx_v], rows_v, sem).wait()
    pltpu.sync_copy(rows_v, out_hbm.at[pl.ds(base, rows_per_w)])


def _make_sc_gather():
    info = plsc.get_sparse_core_info()
    rows_per_w = _N_ROWS // (info.num_cores * info.num_subcores)
    mesh = plsc.VectorSubcoreMesh(core_axis_name="c", subcore_axis_name="s")
    return pl.kernel(
        _sc_gather_body,
        out_type=jax.ShapeDtypeStruct((_N_ROWS, _D_PAD), jnp.float32),
        mesh=mesh,
        scratch_types=[
            pltpu.VMEM((rows_per_w,), jnp.int32),
            pltpu.VMEM((rows_per_w, _D_PAD), jnp.float32),
            pltpu.SemaphoreType.DMA,
        ],
    )


def kernel(inputs, embeddings):
    x = inputs.reshape(_N_ROWS, _EMB_DIM)
    idx3, loss2 = _argmin_call(x, embeddings + embeddings)
    idx = idx3.reshape(_N_ROWS)
    et = jnp.pad(embeddings.T, ((0, 0), (0, _D_PAD - _EMB_DIM)))
    q = _make_sc_gather()(et, idx)
    out = q[:, :_EMB_DIM].reshape(16, 8192, 1)
    return (out, loss2[0, 0], 0.0)
